# Initial kernel scaffold; baseline (speedup 1.0000x reference)
import jax
import jax.numpy as jnp
from jax.experimental import pallas as pl


def kernel(x, edge_index, W1, b1, W_mu, b_mu, W_ls, b_ls):
    raise NotImplementedError("write your pallas kernel here")


# speedup vs baseline: 23.7041x; 23.7041x over previous
import functools

import jax
import jax.numpy as jnp
from jax import lax
from jax.experimental import pallas as pl
from jax.experimental.pallas import tpu as pltpu
from jax.experimental.pallas import tpu_sc as plsc

N = 10000
E = 320000
D_IN = 128
D_HID = 128
D_OUT = 64

NC = 2
NS = 16
NW = NC * NS
EPW = E // NW
CH = 80
NCH = EPW // CH
RPT = 640
NPAD = NS * RPT

_mesh = plsc.VectorSubcoreMesh(core_axis_name="c", subcore_axis_name="s")


@functools.partial(
    pl.kernel,
    out_type=jax.ShapeDtypeStruct((NC, NPAD, 16), jnp.float32),
    mesh=_mesh,
    scratch_types=[
        pltpu.VMEM((NCH, CH), jnp.int32),
        pltpu.VMEM((CH, 16), jnp.float32),
        pltpu.VMEM((CH, 16), jnp.float32),
        pltpu.VMEM_SHARED((NPAD, 16), jnp.float32),
    ],
)
def _sc_deg(dst_hbm, out_hbm, dst_v, ones_v, zero_v, acc_sh):
    c = lax.axis_index("c")
    s = lax.axis_index("s")
    wid = c * NS + s
    row0 = s * RPT

    def fill(i, carry):
        ones_v[i, :] = jnp.ones((16,), jnp.float32)
        zero_v[i, :] = jnp.zeros((16,), jnp.float32)
        return carry

    lax.fori_loop(0, CH, fill, 0)

    def zero_acc(k, carry):
        pltpu.sync_copy(zero_v, acc_sh.at[pl.ds(row0 + k * CH, CH)])
        return carry

    lax.fori_loop(0, RPT // CH, zero_acc, 0)
    pltpu.sync_copy(dst_hbm.at[wid], dst_v)
    plsc.subcore_barrier()

    def step(j, carry):
        pltpu.sync_copy(ones_v, acc_sh.at[dst_v.at[j]], add=True)
        return carry

    lax.fori_loop(0, NCH, step, 0)
    plsc.subcore_barrier()
    pltpu.sync_copy(acc_sh.at[pl.ds(row0, RPT)], out_hbm.at[c, pl.ds(row0, RPT)])


@functools.partial(
    pl.kernel,
    out_type=jax.ShapeDtypeStruct((NC, NPAD, D_HID), jnp.float32),
    mesh=_mesh,
    scratch_types=[
        pltpu.VMEM((NCH, CH), jnp.int32),
        pltpu.VMEM((NCH, CH), jnp.int32),
        pltpu.VMEM((CH, D_HID), jnp.float32),
        pltpu.VMEM_SHARED((NPAD, D_HID), jnp.float32),
        pltpu.SemaphoreType.DMA,
    ],
)
def _sc_agg(zs_hbm, src_hbm, dst_hbm, out_hbm, src_v, dst_v, rows_v, acc_sh, sem):
    c = lax.axis_index("c")
    s = lax.axis_index("s")
    wid = c * NS + s
    row0 = s * RPT

    def zero_rows(i, carry):
        for j in range(D_HID // 16):
            rows_v[i, pl.ds(j * 16, 16)] = jnp.zeros((16,), jnp.float32)
        return carry

    lax.fori_loop(0, CH, zero_rows, 0)

    def zero_acc(k, carry):
        pltpu.sync_copy(rows_v, acc_sh.at[pl.ds(row0 + k * CH, CH)])
        return carry

    lax.fori_loop(0, RPT // CH, zero_acc, 0)
    pltpu.sync_copy(src_hbm.at[wid], src_v)
    pltpu.sync_copy(dst_hbm.at[wid], dst_v)
    plsc.subcore_barrier()

    def step(j, carry):
        pltpu.async_copy(zs_hbm.at[src_v.at[j]], rows_v, sem).wait()
        pltpu.sync_copy(rows_v, acc_sh.at[dst_v.at[j]], add=True)
        return carry

    lax.fori_loop(0, NCH, step, 0)
    plsc.subcore_barrier()
    pltpu.sync_copy(acc_sh.at[pl.ds(row0, RPT)], out_hbm.at[c, pl.ds(row0, RPT)])


def _tc_mm_body(x_ref, w_ref, o_ref):
    o_ref[...] = jnp.dot(x_ref[...], w_ref[...], preferred_element_type=jnp.float32)


_tc_mm = pl.pallas_call(
    _tc_mm_body,
    out_shape=jax.ShapeDtypeStruct((N, D_HID), jnp.float32),
)


def _tc_prep_body(degp_ref, xw_ref, zs_ref, dis_ref):
    deg = degp_ref[0, :N, 0:1] + degp_ref[1, :N, 0:1] + 1.0
    dis = lax.rsqrt(deg)
    dis_ref[...] = dis
    zs_ref[...] = xw_ref[...] * dis


_tc_prep = pl.pallas_call(
    _tc_prep_body,
    out_shape=(
        jax.ShapeDtypeStruct((N, D_HID), jnp.float32),
        jax.ShapeDtypeStruct((N, 1), jnp.float32),
    ),
)


def _tc_mid_body(sp_ref, zs1_ref, dis_ref, b1_ref, zs2_ref):
    dis = dis_ref[...]
    agg = (sp_ref[0, :N, :] + sp_ref[1, :N, :] + zs1_ref[...]) * dis
    h = jnp.maximum(agg + b1_ref[...], 0.0)
    zs2_ref[...] = h * dis


_tc_mid = pl.pallas_call(
    _tc_mid_body,
    out_shape=jax.ShapeDtypeStruct((N, D_HID), jnp.float32),
)


def _tc_out_body(sp_ref, zs2_ref, dis_ref, wmu_ref, bmu_ref, wls_ref, bls_ref,
                 mu_ref, ls_ref):
    agg = (sp_ref[0, :N, :] + sp_ref[1, :N, :] + zs2_ref[...]) * dis_ref[...]
    mu_ref[...] = jnp.dot(agg, wmu_ref[...],
                          preferred_element_type=jnp.float32) + bmu_ref[...]
    ls_ref[...] = jnp.dot(agg, wls_ref[...],
                          preferred_element_type=jnp.float32) + bls_ref[...]


_tc_out = pl.pallas_call(
    _tc_out_body,
    out_shape=(
        jax.ShapeDtypeStruct((N, D_OUT), jnp.float32),
        jax.ShapeDtypeStruct((N, D_OUT), jnp.float32),
    ),
)


def kernel(x, edge_index, W1, b1, W_mu, b_mu, W_ls, b_ls):
    src = edge_index[0].reshape(NW, NCH, CH)
    dst = edge_index[1].reshape(NW, NCH, CH)
    degp = _sc_deg(dst)
    xw = _tc_mm(x, W1)
    zs1, dis = _tc_prep(degp, xw)
    s1 = _sc_agg(zs1, src, dst)
    zs2 = _tc_mid(s1, zs1, dis, b1.reshape(1, D_HID))
    s2 = _sc_agg(zs2, src, dst)
    mu, ls = _tc_out(s2, zs2, dis, W_mu, b_mu.reshape(1, D_OUT),
                     W_ls, b_ls.reshape(1, D_OUT))
    return (mu, ls)


# speedup vs baseline: 29.5503x; 1.2466x over previous
import functools

import jax
import jax.numpy as jnp
from jax import lax
from jax.experimental import pallas as pl
from jax.experimental.pallas import tpu as pltpu
from jax.experimental.pallas import tpu_sc as plsc

N = 10000
E = 320000
D_IN = 128
D_HID = 128
D_OUT = 64

NC = 2
NS = 16
NW = NC * NS
EPW = E // NW
CH = 80
NCH = EPW // CH
RPT = 640
NPAD = NS * RPT

_mesh = plsc.VectorSubcoreMesh(core_axis_name="c", subcore_axis_name="s")


@functools.partial(
    pl.kernel,
    out_type=jax.ShapeDtypeStruct((NC, NPAD, 16), jnp.float32),
    mesh=_mesh,
    scratch_types=[
        pltpu.VMEM((NCH, CH), jnp.int32),
        pltpu.VMEM((CH, 16), jnp.float32),
        pltpu.VMEM((CH, 16), jnp.float32),
        pltpu.VMEM_SHARED((NPAD, 16), jnp.float32),
    ],
)
def _sc_deg(dst_hbm, out_hbm, dst_v, ones_v, zero_v, acc_sh):
    c = lax.axis_index("c")
    s = lax.axis_index("s")
    wid = c * NS + s
    row0 = s * RPT

    def fill(i, carry):
        ones_v[i, :] = jnp.ones((16,), jnp.float32)
        zero_v[i, :] = jnp.zeros((16,), jnp.float32)
        return carry

    lax.fori_loop(0, CH, fill, 0)

    def zero_acc(k, carry):
        pltpu.sync_copy(zero_v, acc_sh.at[pl.ds(row0 + k * CH, CH)])
        return carry

    lax.fori_loop(0, RPT // CH, zero_acc, 0)
    pltpu.sync_copy(dst_hbm.at[wid], dst_v)
    plsc.subcore_barrier()

    def step(j, carry):
        pltpu.sync_copy(ones_v, acc_sh.at[dst_v.at[j]], add=True)
        return carry

    lax.fori_loop(0, NCH, step, 0)
    plsc.subcore_barrier()
    pltpu.sync_copy(acc_sh.at[pl.ds(row0, RPT)], out_hbm.at[c, pl.ds(row0, RPT)])


@functools.partial(
    pl.kernel,
    out_type=jax.ShapeDtypeStruct((NC, NPAD, D_HID), jnp.float32),
    mesh=_mesh,
    scratch_types=[
        pltpu.VMEM((EPW,), jnp.int32),
        pltpu.VMEM((NCH, CH), jnp.int32),
        pltpu.VMEM((CH, D_HID), jnp.float32),
        pltpu.VMEM((CH, D_HID), jnp.float32),
        pltpu.VMEM_SHARED((NPAD, D_HID), jnp.float32),
        pltpu.SemaphoreType.DMA,
        pltpu.SemaphoreType.DMA,
    ],
)
def _sc_agg(zs_hbm, src_hbm, dst_hbm, out_hbm, src_v, dst_v, rows_a, rows_b,
            acc_sh, sem_a, sem_b):
    c = lax.axis_index("c")
    s = lax.axis_index("s")
    wid = c * NS + s
    row0 = s * RPT

    def zero_rows(i, carry):
        for j in range(D_HID // 16):
            rows_a[i, pl.ds(j * 16, 16)] = jnp.zeros((16,), jnp.float32)
        return carry

    lax.fori_loop(0, CH, zero_rows, 0)

    def zero_acc(k, carry):
        pltpu.sync_copy(rows_a, acc_sh.at[pl.ds(row0 + k * CH, CH)])
        return carry

    lax.fori_loop(0, RPT // CH, zero_acc, 0)
    pltpu.sync_copy(src_hbm.at[wid], src_v)
    pltpu.sync_copy(dst_hbm.at[wid], dst_v)
    plsc.subcore_barrier()

    def sidx(j):
        return src_v.at[pl.ds(j * CH, CH)]

    pltpu.make_async_copy(zs_hbm.at[sidx(0)], rows_a, sem_a).start()

    def pair(k, carry):
        j0 = 2 * k
        pltpu.make_async_copy(zs_hbm.at[sidx(j0)], rows_a, sem_a).wait()
        pltpu.make_async_copy(zs_hbm.at[sidx(j0 + 1)], rows_b, sem_b).start()
        pltpu.sync_copy(rows_a, acc_sh.at[dst_v.at[j0]], add=True)
        pltpu.make_async_copy(zs_hbm.at[sidx(j0 + 1)], rows_b, sem_b).wait()
        pltpu.make_async_copy(zs_hbm.at[sidx(j0 + 2)], rows_a, sem_a).start()
        pltpu.sync_copy(rows_b, acc_sh.at[dst_v.at[j0 + 1]], add=True)
        return carry

    lax.fori_loop(0, NCH // 2, pair, 0)
    pltpu.make_async_copy(zs_hbm.at[sidx(NCH - 1)], rows_a, sem_a).wait()
    pltpu.sync_copy(rows_a, acc_sh.at[dst_v.at[NCH - 1]], add=True)
    plsc.subcore_barrier()
    pltpu.sync_copy(acc_sh.at[pl.ds(row0, RPT)], out_hbm.at[c, pl.ds(row0, RPT)])


def _tc_mm_body(x_ref, w_ref, o_ref):
    o_ref[...] = jnp.dot(x_ref[...], w_ref[...], preferred_element_type=jnp.float32)


_tc_mm = pl.pallas_call(
    _tc_mm_body,
    out_shape=jax.ShapeDtypeStruct((N, D_HID), jnp.float32),
)


def _tc_prep_body(degp_ref, xw_ref, zs_ref, dis_ref):
    deg = degp_ref[0, :N, 0:1] + degp_ref[1, :N, 0:1] + 1.0
    dis = lax.rsqrt(deg)
    dis_ref[...] = dis
    zs_ref[...] = xw_ref[...] * dis


_tc_prep = pl.pallas_call(
    _tc_prep_body,
    out_shape=(
        jax.ShapeDtypeStruct((N, D_HID), jnp.float32),
        jax.ShapeDtypeStruct((N, 1), jnp.float32),
    ),
)


def _tc_mid_body(sp_ref, zs1_ref, dis_ref, b1_ref, zs2_ref):
    dis = dis_ref[...]
    agg = (sp_ref[0, :N, :] + sp_ref[1, :N, :] + zs1_ref[...]) * dis
    h = jnp.maximum(agg + b1_ref[...], 0.0)
    zs2_ref[...] = h * dis


_tc_mid = pl.pallas_call(
    _tc_mid_body,
    out_shape=jax.ShapeDtypeStruct((N, D_HID), jnp.float32),
)


def _tc_out_body(sp_ref, zs2_ref, dis_ref, wmu_ref, bmu_ref, wls_ref, bls_ref,
                 mu_ref, ls_ref):
    agg = (sp_ref[0, :N, :] + sp_ref[1, :N, :] + zs2_ref[...]) * dis_ref[...]
    mu_ref[...] = jnp.dot(agg, wmu_ref[...],
                          preferred_element_type=jnp.float32) + bmu_ref[...]
    ls_ref[...] = jnp.dot(agg, wls_ref[...],
                          preferred_element_type=jnp.float32) + bls_ref[...]


_tc_out = pl.pallas_call(
    _tc_out_body,
    out_shape=(
        jax.ShapeDtypeStruct((N, D_OUT), jnp.float32),
        jax.ShapeDtypeStruct((N, D_OUT), jnp.float32),
    ),
)


def kernel(x, edge_index, W1, b1, W_mu, b_mu, W_ls, b_ls):
    src = edge_index[0].reshape(NW, EPW)
    dst = edge_index[1].reshape(NW, NCH, CH)
    degp = _sc_deg(dst)
    xw = _tc_mm(x, W1)
    zs1, dis = _tc_prep(degp, xw)
    s1 = _sc_agg(zs1, src, dst)
    zs2 = _tc_mid(s1, zs1, dis, b1.reshape(1, D_HID))
    s2 = _sc_agg(zs2, src, dst)
    mu, ls = _tc_out(s2, zs2, dis, W_mu, b_mu.reshape(1, D_OUT),
                     W_ls, b_ls.reshape(1, D_OUT))
    return (mu, ls)
